# baseline (speedup 1.0000x reference)
import jax
import jax.numpy as jnp
from jax import lax
from jax.experimental import pallas as pl
from jax.experimental.pallas import tpu as pltpu
from jax.experimental.pallas import tpu_sc as plsc

NB = 16
L = 4098
D = 256
NOPS = 4
DOP = 64
T = 32768
HALF = L // 2
C = 104
CZ = 64
NV = D // 16
PAD_LO = C
PAD_HI = L
TP = T + PAD_LO + PAD_HI


def _sc_body(e_hbm, wrow_hbm, starts_hbm, lens_hbm, out_hbm,
             in0, in1, ob0, ob1, zbuf, wrow_v, starts_v, lens_v,
             sin0, sin1, sout0, sout1, sz):
    cid = lax.axis_index("c")
    sid = lax.axis_index("s")
    b = sid
    half = (cid + sid) % 2
    p0 = half * HALF
    row_base = b * L

    pltpu.sync_copy(wrow_hbm, wrow_v)
    pltpu.sync_copy(starts_hbm, starts_v)
    pltpu.sync_copy(lens_hbm, lens_v)

    lane = lax.broadcasted_iota(jnp.int32, (16,), 0)
    sel = (lane == b).astype(jnp.int32)
    st_b = jnp.sum(starts_v[...] * sel)
    len_b = jnp.sum(lens_v[...] * sel)

    wregs = [wrow_v[pl.ds(v * 16, 16)] for v in range(NV)]
    zv = jnp.zeros((16,), jnp.float32)
    ones_v = zv + 1.0
    twos_v = zv + 2.0

    zend = p0 + HALF
    bend = jnp.clip(len_b + 2, p0, zend)
    nz = jnp.maximum(zend - bend, 0) // CZ
    b_end = zend - nz * CZ
    nt = (jnp.maximum(b_end - p0, 0) + C - 1) // C

    inbufs = (in0, in1)
    obufs = (ob0, ob1)
    sins = (sin0, sin1)
    souts = (sout0, sout1)

    def chunk_start(j):
        return jnp.maximum(jnp.minimum(p0 + j * C, b_end - C), p0)

    def fire_in(j, slot):
        s_j = chunk_start(j)
        t0p = st_b + s_j - 1 + PAD_LO
        pltpu.async_copy(e_hbm.at[:, pl.ds(t0p, C), :], inbufs[slot],
                         sins[slot])

    @pl.when(nt >= 1)
    def _():
        fire_in(0, 0)

    @pl.when(nt >= 2)
    def _():
        fire_in(1, 1)

    @plsc.parallel_loop(0, CZ, unroll=4)
    def _(i):
        for v in range(NV):
            zbuf[i, pl.ds(v * 16, 16)] = zv

    def zfire(j, carry):
        s = zend - (j + 1) * CZ
        pltpu.async_copy(zbuf, out_hbm.at[pl.ds(row_base + s, CZ), :], sz)
        return carry
    lax.fori_loop(0, nz, zfire, 0)

    def do_chunk(j, slot):
        ib = inbufs[slot]
        ob = obufs[slot]
        s_j = chunk_start(j)
        t0p = st_b + s_j - 1 + PAD_LO
        pltpu.make_async_copy(e_hbm.at[:, pl.ds(t0p, C), :], ib,
                              sins[slot]).wait()

        @pl.when(j >= 2)
        def _():
            pltpu.make_async_copy(
                ob, out_hbm.at[pl.ds(row_base + s_j, C), :],
                souts[slot]).wait()

        @plsc.parallel_loop(0, C, unroll=4)
        def _(i):
            for v in range(NV):
                x = ib[v // 4, i, pl.ds((v % 4) * 16, 16)]
                ob[i, pl.ds(v * 16, 16)] = x * wregs[v]

        zs = jnp.clip(len_b + 2 - s_j, 0, C)

        @plsc.parallel_loop(0, C - zs, unroll=2)
        def _(k):
            i = zs + k
            for v in range(NV):
                ob[i, pl.ds(v * 16, 16)] = zv

        @pl.when((len_b + 1 >= s_j) & (len_b + 1 < s_j + C))
        def _():
            i = len_b + 1 - s_j
            for v in range(NV):
                ob[i, pl.ds(v * 16, 16)] = twos_v

        @pl.when(s_j == 0)
        def _():
            for v in range(NV):
                ob[0, pl.ds(v * 16, 16)] = ones_v

        pltpu.async_copy(ob, out_hbm.at[pl.ds(row_base + s_j, C), :],
                         souts[slot])

        @pl.when(j + 2 < nt)
        def _():
            fire_in(j + 2, slot)

    def pair(jj, carry):
        j0 = 2 * jj

        @pl.when(j0 < nt)
        def _():
            do_chunk(j0, 0)

        @pl.when(j0 + 1 < nt)
        def _():
            do_chunk(j0 + 1, 1)
        return carry

    lax.fori_loop(0, (nt + 1) // 2, pair, 0)

    def zdrain(j, carry):
        pltpu.make_async_copy(zbuf, out_hbm.at[pl.ds(row_base + p0, CZ), :],
                              sz).wait()
        return carry
    lax.fori_loop(0, nz, zdrain, 0)

    @pl.when(nt >= 1)
    def _():
        s_last = chunk_start(nt - 1)

        @pl.when((nt - 1) % 2 == 0)
        def _():
            pltpu.make_async_copy(
                ob0, out_hbm.at[pl.ds(row_base + s_last, C), :],
                sout0).wait()

        @pl.when((nt - 1) % 2 == 1)
        def _():
            pltpu.make_async_copy(
                ob1, out_hbm.at[pl.ds(row_base + s_last, C), :],
                sout1).wait()

    @pl.when(nt >= 2)
    def _():
        s_prev = chunk_start(nt - 2)

        @pl.when((nt - 2) % 2 == 0)
        def _():
            pltpu.make_async_copy(
                ob0, out_hbm.at[pl.ds(row_base + s_prev, C), :],
                sout0).wait()

        @pl.when((nt - 2) % 2 == 1)
        def _():
            pltpu.make_async_copy(
                ob1, out_hbm.at[pl.ds(row_base + s_prev, C), :],
                sout1).wait()


def kernel(token_embeds, weights, cu_seqlens):
    w = jax.nn.softmax(weights, axis=-1)
    wrow = jnp.repeat(w, DOP)
    starts = cu_seqlens[:NB]
    lens = cu_seqlens[1:] - cu_seqlens[:-1]
    e_pad = jnp.pad(token_embeds, ((0, 0), (PAD_LO, PAD_HI), (0, 0)))
    mesh = plsc.VectorSubcoreMesh(core_axis_name="c", subcore_axis_name="s")
    run = pl.kernel(
        _sc_body,
        mesh=mesh,
        compiler_params=pltpu.CompilerParams(
            use_tc_tiling_on_sc=False, needs_layout_passes=False),
        out_type=jax.ShapeDtypeStruct((NB * L, D), jnp.float32),
        scratch_types=[
            pltpu.VMEM((NOPS, C, DOP), jnp.float32),
            pltpu.VMEM((NOPS, C, DOP), jnp.float32),
            pltpu.VMEM((C, D), jnp.float32),
            pltpu.VMEM((C, D), jnp.float32),
            pltpu.VMEM((CZ, D), jnp.float32),
            pltpu.VMEM((D,), jnp.float32),
            pltpu.VMEM((16,), jnp.int32),
            pltpu.VMEM((16,), jnp.int32),
            pltpu.SemaphoreType.DMA,
            pltpu.SemaphoreType.DMA,
            pltpu.SemaphoreType.DMA,
            pltpu.SemaphoreType.DMA,
            pltpu.SemaphoreType.DMA,
        ],
    )
    out = run(e_pad, wrow, starts, lens)
    return out.reshape(NB, L, D)


# speedup vs baseline: 2.5334x; 2.5334x over previous
import jax
import jax.numpy as jnp
from jax import lax
from jax.experimental import pallas as pl
from jax.experimental.pallas import tpu as pltpu
from jax.experimental.pallas import tpu_sc as plsc

NB = 16
L = 4098
D = 256
NOPS = 4
DOP = 64
T = 32768
HALF = L // 2
C = 104
CZ = 64
NV = D // 16


def _sc_body(e_hbm, wrow_hbm, starts_hbm, lens_hbm, out_hbm,
             in0, in1, ob0, ob1, zbuf, wrow_v, starts_v, lens_v,
             sin0, sin1, sout0, sout1, sz):
    cid = lax.axis_index("c")
    sid = lax.axis_index("s")
    b = sid
    half = (cid + sid) % 2
    p0 = half * HALF
    row_base = b * L

    pltpu.sync_copy(wrow_hbm, wrow_v)
    pltpu.sync_copy(starts_hbm, starts_v)
    pltpu.sync_copy(lens_hbm, lens_v)

    lane = lax.broadcasted_iota(jnp.int32, (16,), 0)
    sel = (lane == b).astype(jnp.int32)
    st_b = jnp.sum(starts_v[...] * sel)
    len_b = jnp.sum(lens_v[...] * sel)

    wregs = [wrow_v[pl.ds(v * 16, 16)] for v in range(NV)]
    zv = jnp.zeros((16,), jnp.float32)
    ones_v = zv + 1.0
    twos_v = zv + 2.0

    zend = p0 + HALF
    bend = jnp.clip(len_b + 2, p0, zend)
    nz = jnp.maximum(zend - bend, 0) // CZ
    b_end = zend - nz * CZ
    nt = (jnp.maximum(b_end - p0, 0) + C - 1) // C

    inbufs = (in0, in1)
    obufs = (ob0, ob1)
    sins = (sin0, sin1)
    souts = (sout0, sout1)

    def chunk_start(j):
        return jnp.maximum(jnp.minimum(p0 + j * C, b_end - C), p0)

    def window_start(s_j):
        return jnp.clip(st_b + s_j - 1, 0, T - C)

    def fire_in(j, slot):
        s_j = chunk_start(j)
        t0c = window_start(s_j)
        pltpu.async_copy(e_hbm.at[:, pl.ds(t0c, C), :], inbufs[slot],
                         sins[slot])

    @pl.when(nt >= 1)
    def _():
        fire_in(0, 0)

    @pl.when(nt >= 2)
    def _():
        fire_in(1, 1)

    @plsc.parallel_loop(0, CZ, unroll=4)
    def _(i):
        for v in range(NV):
            zbuf[i, pl.ds(v * 16, 16)] = zv

    def zfire(j, carry):
        s = zend - (j + 1) * CZ
        pltpu.async_copy(zbuf, out_hbm.at[pl.ds(row_base + s, CZ), :], sz)
        return carry
    lax.fori_loop(0, nz, zfire, 0)

    def do_chunk(j, slot):
        ib = inbufs[slot]
        ob = obufs[slot]
        s_j = chunk_start(j)
        t0c = window_start(s_j)
        delta = st_b + s_j - 1 - t0c
        pltpu.make_async_copy(e_hbm.at[:, pl.ds(t0c, C), :], ib,
                              sins[slot]).wait()

        @pl.when(j >= 2)
        def _():
            pltpu.make_async_copy(
                ob, out_hbm.at[pl.ds(row_base + s_j, C), :],
                souts[slot]).wait()

        @pl.when(delta == 0)
        def _():
            @plsc.parallel_loop(0, C, unroll=4)
            def _(i):
                for v in range(NV):
                    x = ib[v // 4, i, pl.ds((v % 4) * 16, 16)]
                    ob[i, pl.ds(v * 16, 16)] = x * wregs[v]

        @pl.when(delta != 0)
        def _():
            @plsc.parallel_loop(0, C, unroll=4)
            def _(i):
                rp = jnp.clip(i + delta, 0, C - 1)
                for v in range(NV):
                    x = ib[v // 4, rp, pl.ds((v % 4) * 16, 16)]
                    ob[i, pl.ds(v * 16, 16)] = x * wregs[v]

        zs = jnp.clip(len_b + 2 - s_j, 0, C)

        @plsc.parallel_loop(0, C - zs, unroll=2)
        def _(k):
            i = zs + k
            for v in range(NV):
                ob[i, pl.ds(v * 16, 16)] = zv

        @pl.when((len_b + 1 >= s_j) & (len_b + 1 < s_j + C))
        def _():
            i = len_b + 1 - s_j
            for v in range(NV):
                ob[i, pl.ds(v * 16, 16)] = twos_v

        @pl.when(s_j == 0)
        def _():
            for v in range(NV):
                ob[0, pl.ds(v * 16, 16)] = ones_v

        pltpu.async_copy(ob, out_hbm.at[pl.ds(row_base + s_j, C), :],
                         souts[slot])

        @pl.when(j + 2 < nt)
        def _():
            fire_in(j + 2, slot)

    def pair(jj, carry):
        j0 = 2 * jj

        @pl.when(j0 < nt)
        def _():
            do_chunk(j0, 0)

        @pl.when(j0 + 1 < nt)
        def _():
            do_chunk(j0 + 1, 1)
        return carry

    lax.fori_loop(0, (nt + 1) // 2, pair, 0)

    def zdrain(j, carry):
        pltpu.make_async_copy(zbuf, out_hbm.at[pl.ds(row_base + p0, CZ), :],
                              sz).wait()
        return carry
    lax.fori_loop(0, nz, zdrain, 0)

    @pl.when(nt >= 1)
    def _():
        s_last = chunk_start(nt - 1)

        @pl.when((nt - 1) % 2 == 0)
        def _():
            pltpu.make_async_copy(
                ob0, out_hbm.at[pl.ds(row_base + s_last, C), :],
                sout0).wait()

        @pl.when((nt - 1) % 2 == 1)
        def _():
            pltpu.make_async_copy(
                ob1, out_hbm.at[pl.ds(row_base + s_last, C), :],
                sout1).wait()

    @pl.when(nt >= 2)
    def _():
        s_prev = chunk_start(nt - 2)

        @pl.when((nt - 2) % 2 == 0)
        def _():
            pltpu.make_async_copy(
                ob0, out_hbm.at[pl.ds(row_base + s_prev, C), :],
                sout0).wait()

        @pl.when((nt - 2) % 2 == 1)
        def _():
            pltpu.make_async_copy(
                ob1, out_hbm.at[pl.ds(row_base + s_prev, C), :],
                sout1).wait()


def kernel(token_embeds, weights, cu_seqlens):
    w = jax.nn.softmax(weights, axis=-1)
    wrow = jnp.repeat(w, DOP)
    starts = cu_seqlens[:NB]
    lens = cu_seqlens[1:] - cu_seqlens[:-1]
    mesh = plsc.VectorSubcoreMesh(core_axis_name="c", subcore_axis_name="s")
    run = pl.kernel(
        _sc_body,
        mesh=mesh,
        compiler_params=pltpu.CompilerParams(
            use_tc_tiling_on_sc=False, needs_layout_passes=False),
        out_type=jax.ShapeDtypeStruct((NB * L, D), jnp.float32),
        scratch_types=[
            pltpu.VMEM((NOPS, C, DOP), jnp.float32),
            pltpu.VMEM((NOPS, C, DOP), jnp.float32),
            pltpu.VMEM((C, D), jnp.float32),
            pltpu.VMEM((C, D), jnp.float32),
            pltpu.VMEM((CZ, D), jnp.float32),
            pltpu.VMEM((D,), jnp.float32),
            pltpu.VMEM((16,), jnp.int32),
            pltpu.VMEM((16,), jnp.int32),
            pltpu.SemaphoreType.DMA,
            pltpu.SemaphoreType.DMA,
            pltpu.SemaphoreType.DMA,
            pltpu.SemaphoreType.DMA,
            pltpu.SemaphoreType.DMA,
        ],
    )
    out = run(token_embeds, wrow, starts, lens)
    return out.reshape(NB, L, D)


# speedup vs baseline: 4.5666x; 1.8026x over previous
import jax
import jax.numpy as jnp
from jax import lax
from jax.experimental import pallas as pl
from jax.experimental.pallas import tpu as pltpu
from jax.experimental.pallas import tpu_sc as plsc

NB = 16
L = 4098
D = 256
NOPS = 4
DOP = 64
T = 32768
HALF = L // 2
C = 104
CZ = 64
NV = D // 16


def _sc_body(e_hbm, wrow_hbm, starts_hbm, lens_hbm, out_hbm,
             in0, in1, ob0, ob1, zbuf, wrow_v, starts_v, lens_v,
             sin0, sin1, sout0, sout1, sz):
    cid = lax.axis_index("c")
    sid = lax.axis_index("s")
    b = sid
    half = (cid + sid) % 2
    p0 = half * HALF
    bhi = b // 8
    blo = b % 8

    pltpu.sync_copy(wrow_hbm, wrow_v)
    pltpu.sync_copy(starts_hbm, starts_v)
    pltpu.sync_copy(lens_hbm, lens_v)

    lane = lax.broadcasted_iota(jnp.int32, (16,), 0)
    sel = (lane == b).astype(jnp.int32)
    st_b = jnp.sum(starts_v[...] * sel)
    len_b = jnp.sum(lens_v[...] * sel)

    wregs = [wrow_v[pl.ds(v * 16, 16)] for v in range(NV)]
    zv = jnp.zeros((16,), jnp.float32)
    ones_v = zv + 1.0
    twos_v = zv + 2.0

    zend = p0 + HALF
    bend = jnp.clip(len_b + 2, p0, zend)
    nz = jnp.maximum(zend - bend, 0) // CZ
    b_end = zend - nz * CZ
    nt = (jnp.maximum(b_end - p0, 0) + C - 1) // C

    inbufs = (in0, in1)
    obufs = (ob0, ob1)
    sins = (sin0, sin1)
    souts = (sout0, sout1)

    def chunk_start(j):
        return jnp.maximum(jnp.minimum(p0 + j * C, b_end - C), p0)

    def window_start(s_j):
        return jnp.clip(st_b + s_j - 1, 0, T - C)

    def fire_in(j, slot):
        s_j = chunk_start(j)
        t0c = window_start(s_j)
        pltpu.async_copy(e_hbm.at[:, pl.ds(t0c, C), :], inbufs[slot],
                         sins[slot])

    @pl.when(nt >= 1)
    def _():
        fire_in(0, 0)

    @pl.when(nt >= 2)
    def _():
        fire_in(1, 1)

    @plsc.parallel_loop(0, CZ, unroll=4)
    def _(i):
        for v in range(NV):
            zbuf[i, v // 8, pl.ds((v % 8) * 16, 16)] = zv

    def zfire(j, carry):
        s = zend - (j + 1) * CZ
        pltpu.async_copy(zbuf, out_hbm.at[pl.ds(s, CZ), bhi, :, blo, :], sz)
        return carry
    lax.fori_loop(0, nz, zfire, 0)

    def do_chunk(j, slot):
        ib = inbufs[slot]
        ob = obufs[slot]
        s_j = chunk_start(j)
        t0c = window_start(s_j)
        delta = st_b + s_j - 1 - t0c
        pltpu.make_async_copy(e_hbm.at[:, pl.ds(t0c, C), :], ib,
                              sins[slot]).wait()

        @pl.when(j >= 2)
        def _():
            pltpu.make_async_copy(
                ob, out_hbm.at[pl.ds(s_j, C), bhi, :, blo, :],
                souts[slot]).wait()

        @pl.when(delta == 0)
        def _():
            @plsc.parallel_loop(0, C, unroll=4)
            def _(i):
                for v in range(NV):
                    x = ib[v // 4, i, pl.ds((v % 4) * 16, 16)]
                    ob[i, v // 8, pl.ds((v % 8) * 16, 16)] = x * wregs[v]

        @pl.when(delta != 0)
        def _():
            @plsc.parallel_loop(0, C, unroll=4)
            def _(i):
                rp = jnp.clip(i + delta, 0, C - 1)
                for v in range(NV):
                    x = ib[v // 4, rp, pl.ds((v % 4) * 16, 16)]
                    ob[i, v // 8, pl.ds((v % 8) * 16, 16)] = x * wregs[v]

        zs = jnp.clip(len_b + 2 - s_j, 0, C)

        @plsc.parallel_loop(0, C - zs, unroll=2)
        def _(k):
            i = zs + k
            for v in range(NV):
                ob[i, v // 8, pl.ds((v % 8) * 16, 16)] = zv

        @pl.when((len_b + 1 >= s_j) & (len_b + 1 < s_j + C))
        def _():
            i = len_b + 1 - s_j
            for v in range(NV):
                ob[i, v // 8, pl.ds((v % 8) * 16, 16)] = twos_v

        @pl.when(s_j == 0)
        def _():
            for v in range(NV):
                ob[0, v // 8, pl.ds((v % 8) * 16, 16)] = ones_v

        pltpu.async_copy(ob, out_hbm.at[pl.ds(s_j, C), bhi, :, blo, :],
                         souts[slot])

        @pl.when(j + 2 < nt)
        def _():
            fire_in(j + 2, slot)

    def pair(jj, carry):
        j0 = 2 * jj

        @pl.when(j0 < nt)
        def _():
            do_chunk(j0, 0)

        @pl.when(j0 + 1 < nt)
        def _():
            do_chunk(j0 + 1, 1)
        return carry

    lax.fori_loop(0, (nt + 1) // 2, pair, 0)

    def zdrain(j, carry):
        pltpu.make_async_copy(zbuf, out_hbm.at[pl.ds(p0, CZ), bhi, :, blo, :],
                              sz).wait()
        return carry
    lax.fori_loop(0, nz, zdrain, 0)

    @pl.when(nt >= 1)
    def _():
        s_last = chunk_start(nt - 1)

        @pl.when((nt - 1) % 2 == 0)
        def _():
            pltpu.make_async_copy(
                ob0, out_hbm.at[pl.ds(s_last, C), bhi, :, blo, :],
                sout0).wait()

        @pl.when((nt - 1) % 2 == 1)
        def _():
            pltpu.make_async_copy(
                ob1, out_hbm.at[pl.ds(s_last, C), bhi, :, blo, :],
                sout1).wait()

    @pl.when(nt >= 2)
    def _():
        s_prev = chunk_start(nt - 2)

        @pl.when((nt - 2) % 2 == 0)
        def _():
            pltpu.make_async_copy(
                ob0, out_hbm.at[pl.ds(s_prev, C), bhi, :, blo, :],
                sout0).wait()

        @pl.when((nt - 2) % 2 == 1)
        def _():
            pltpu.make_async_copy(
                ob1, out_hbm.at[pl.ds(s_prev, C), bhi, :, blo, :],
                sout1).wait()


def kernel(token_embeds, weights, cu_seqlens):
    w = jax.nn.softmax(weights, axis=-1)
    wrow = jnp.repeat(w, DOP)
    starts = cu_seqlens[:NB]
    lens = cu_seqlens[1:] - cu_seqlens[:-1]
    mesh = plsc.VectorSubcoreMesh(core_axis_name="c", subcore_axis_name="s")
    run = pl.kernel(
        _sc_body,
        mesh=mesh,
        compiler_params=pltpu.CompilerParams(
            use_tc_tiling_on_sc=False, needs_layout_passes=False),
        out_type=jax.ShapeDtypeStruct((L, 2, 2, 8, 128), jnp.float32),
        scratch_types=[
            pltpu.VMEM((NOPS, C, DOP), jnp.float32),
            pltpu.VMEM((NOPS, C, DOP), jnp.float32),
            pltpu.VMEM((C, 2, 128), jnp.float32),
            pltpu.VMEM((C, 2, 128), jnp.float32),
            pltpu.VMEM((CZ, 2, 128), jnp.float32),
            pltpu.VMEM((D,), jnp.float32),
            pltpu.VMEM((16,), jnp.int32),
            pltpu.VMEM((16,), jnp.int32),
            pltpu.SemaphoreType.DMA,
            pltpu.SemaphoreType.DMA,
            pltpu.SemaphoreType.DMA,
            pltpu.SemaphoreType.DMA,
            pltpu.SemaphoreType.DMA,
        ],
    )
    out = run(token_embeds, wrow, starts, lens)
    return out.transpose(1, 3, 0, 2, 4).reshape(NB, L, D)
